# baseline (speedup 1.0000x reference)
import dataclasses
import functools

import jax
import jax.numpy as jnp
from jax import lax
from jax.experimental import pallas as pl
from jax.experimental.pallas import tpu as pltpu
from jax.experimental.pallas import tpu_sc as plsc

D = 128
LANES = 16
NC, NS = 2, 16
NW = NC * NS


@functools.lru_cache(maxsize=None)
def _w2v_sc_call(B, CP):
    P = CP // 2
    rows_total = B * CP
    b_per_w = B // NW
    c_per_w = rows_total // NW
    p_per_w = c_per_w // 2
    CHP = 8 * P
    CB = 8
    CHC = CB * CP
    n_chunks = b_per_w // CB
    n_red = CHC // LANES
    OB = 16
    mesh = plsc.VectorSubcoreMesh(core_axis_name="c", subcore_axis_name="s")
    cp = pltpu.CompilerParams()
    if "needs_layout_passes" in pltpu.CompilerParams.__dataclass_fields__:
        cp = dataclasses.replace(cp, needs_layout_passes=False)

    @functools.partial(
        pl.kernel,
        mesh=mesh,
        compiler_params=cp,
        out_type=jax.ShapeDtypeStruct((rows_total,), jnp.float32),
        scratch_types=[
            pltpu.VMEM((b_per_w,), jnp.int32),
            pltpu.VMEM((p_per_w,), jnp.int32),
            pltpu.VMEM((p_per_w,), jnp.int32),
            pltpu.VMEM((2, CB, D), jnp.float32),
            pltpu.VMEM((2, CHC, D), jnp.float32),
            pltpu.VMEM((OB * CHC + LANES,), jnp.float32),
            pltpu.SemaphoreType.DMA,
            pltpu.SemaphoreType.DMA,
            pltpu.SemaphoreType.DMA,
            pltpu.SemaphoreType.DMA,
        ],
    )
    def w2v_kernel(wemb, cemb, widx_hbm, pidx_hbm, nidx_hbm, out_hbm,
                   widx_v, pidx_v, nidx_v, wrows_v, crows_v, out_v,
                   sem_w0, sem_w1, sem_c0, sem_c1):
        wid = lax.axis_index("s") * NC + lax.axis_index("c")
        wbase = wid * b_per_w
        cbase = wid * c_per_w
        pbase = wid * p_per_w
        pltpu.sync_copy(widx_hbm.at[pl.ds(wbase, b_per_w)], widx_v)
        pltpu.sync_copy(pidx_hbm.at[pl.ds(pbase, p_per_w)], pidx_v)
        pltpu.sync_copy(nidx_hbm.at[pl.ds(pbase, p_per_w)], nidx_v)
        last_lane = lax.iota(jnp.int32, LANES) == LANES - 1
        sems = ((sem_w0, sem_c0), (sem_w1, sem_c1))

        def fire(k, buf):
            sw, sc2 = sems[buf]
            pltpu.async_copy(
                wemb.at[widx_v.at[pl.ds(k * CB, CB)]], wrows_v.at[buf], sw)
            pltpu.async_copy(
                cemb.at[pidx_v.at[pl.ds(k * CHP, CHP)]],
                crows_v.at[buf].at[pl.ds(0, CHP)], sc2)
            pltpu.async_copy(
                cemb.at[nidx_v.at[pl.ds(k * CHP, CHP)]],
                crows_v.at[buf].at[pl.ds(CHP, CHP)], sc2)

        def wait(buf):
            sw, sc2 = sems[buf]
            pltpu.make_async_copy(
                wemb.at[widx_v.at[pl.ds(0, CB)]], wrows_v.at[buf], sw).wait()
            pltpu.make_async_copy(
                cemb.at[pidx_v.at[pl.ds(0, CHC)]], crows_v.at[buf],
                sc2).wait()

        def compute(g, buf):
            slot = lax.rem(g, OB)
            for b in range(CB):
                w = [wrows_v[buf, b, pl.ds(LANES * j, LANES)] for j in range(8)]

                @plsc.parallel_loop(0, CP, unroll=1)
                def _(c):
                    r = b * P + c + jnp.where(c >= P, CHP - P, 0)
                    acc = w[0] * crows_v[buf, r, pl.ds(0, LANES)]
                    for j in range(1, 8):
                        acc = acc + w[j] * crows_v[buf, r,
                                                   pl.ds(LANES * j, LANES)]
                    total = jnp.cumsum(acc)
                    plsc.store_compressed(
                        out_v.at[pl.ds(slot * CHC + b * CP + c, LANES)],
                        total, mask=last_lane)

            @plsc.parallel_loop(0, n_red, unroll=1)
            def _(q):
                off = slot * CHC + q * LANES
                x = out_v[pl.ds(off, LANES)]
                out_v[pl.ds(off, LANES)] = 1.0 / (1.0 + jnp.exp(-x))

        fire(0, 0)
        fire(1, 1)

        @pl.loop(0, n_chunks - 2, step=2)
        def _(g):
            wait(0)
            compute(g, 0)
            fire(g + 2, 0)
            wait(1)
            compute(g + 1, 1)
            fire(g + 3, 1)

            @pl.when(lax.rem(g + 1, OB) == OB - 1)
            def _():
                pltpu.sync_copy(
                    out_v.at[pl.ds(0, OB * CHC)],
                    out_hbm.at[pl.ds(cbase + (g + 1 - (OB - 1)) * CHC,
                                     OB * CHC)])

        wait(0)
        compute(n_chunks - 2, 0)
        wait(1)
        compute(n_chunks - 1, 1)
        pltpu.sync_copy(
            out_v.at[pl.ds(0, OB * CHC)],
            out_hbm.at[pl.ds(cbase + (n_chunks - OB) * CHC, OB * CHC)])

    return w2v_kernel


def kernel(words, positive_contexts, negative_contexts, word_emb, context_emb):
    B = words.shape[0]
    P = positive_contexts.shape[1]
    N = negative_contexts.shape[1]
    CP = P + N
    out = _w2v_sc_call(B, CP)(
        word_emb, context_emb, words,
        positive_contexts.reshape(B * P), negative_contexts.reshape(B * N))
    out = out.reshape(B, CP)
    return out[:, :P], out[:, P:]


# speedup vs baseline: 1.0303x; 1.0303x over previous
import dataclasses
import functools

import jax
import jax.numpy as jnp
from jax import lax
from jax.experimental import pallas as pl
from jax.experimental.pallas import tpu as pltpu
from jax.experimental.pallas import tpu_sc as plsc

D = 128
LANES = 16
NC, NS = 2, 16
NW = NC * NS


@functools.lru_cache(maxsize=None)
def _w2v_sc_call(B, CP):
    rows_total = B * CP
    b_per_w = B // NW
    c_per_w = rows_total // NW
    CB = 8
    CHC = CB * CP
    n_chunks = b_per_w // CB
    n_red = CHC // LANES
    OB = 16
    mesh = plsc.VectorSubcoreMesh(core_axis_name="c", subcore_axis_name="s")
    cp = pltpu.CompilerParams()
    if "needs_layout_passes" in pltpu.CompilerParams.__dataclass_fields__:
        cp = dataclasses.replace(cp, needs_layout_passes=False)

    @functools.partial(
        pl.kernel,
        mesh=mesh,
        compiler_params=cp,
        out_type=jax.ShapeDtypeStruct((rows_total,), jnp.float32),
        scratch_types=[
            pltpu.VMEM((b_per_w,), jnp.int32),
            pltpu.VMEM((c_per_w,), jnp.int32),
            pltpu.VMEM((2, CB, D), jnp.float32),
            pltpu.VMEM((2, CHC, D), jnp.float32),
            pltpu.VMEM((OB * CHC + LANES,), jnp.float32),
            pltpu.SemaphoreType.DMA,
            pltpu.SemaphoreType.DMA,
            pltpu.SemaphoreType.DMA,
            pltpu.SemaphoreType.DMA,
        ],
    )
    def w2v_kernel(wemb, cemb, widx_hbm, cidx_hbm, out_hbm,
                   widx_v, cidx_v, wrows_v, crows_v, out_v,
                   sem_w0, sem_w1, sem_c0, sem_c1):
        wid = lax.axis_index("s") * NC + lax.axis_index("c")
        wbase = wid * b_per_w
        cbase = wid * c_per_w
        pltpu.sync_copy(widx_hbm.at[pl.ds(wbase, b_per_w)], widx_v)
        pltpu.sync_copy(cidx_hbm.at[pl.ds(cbase, c_per_w)], cidx_v)
        last_lane = lax.iota(jnp.int32, LANES) == LANES - 1
        sems = ((sem_w0, sem_c0), (sem_w1, sem_c1))

        def fire(k, buf):
            sw, sc2 = sems[buf]
            pltpu.async_copy(
                wemb.at[widx_v.at[pl.ds(k * CB, CB)]], wrows_v.at[buf], sw)
            pltpu.async_copy(
                cemb.at[cidx_v.at[pl.ds(k * CHC, CHC)]], crows_v.at[buf], sc2)

        def wait(buf):
            sw, sc2 = sems[buf]
            pltpu.make_async_copy(
                wemb.at[widx_v.at[pl.ds(0, CB)]], wrows_v.at[buf], sw).wait()
            pltpu.make_async_copy(
                cemb.at[cidx_v.at[pl.ds(0, CHC)]], crows_v.at[buf],
                sc2).wait()

        def compute(g, buf):
            slot = lax.rem(g, OB)
            for b in range(CB):
                w = [wrows_v[buf, b, pl.ds(LANES * j, LANES)] for j in range(8)]

                @plsc.parallel_loop(0, CP, unroll=2)
                def _(c):
                    r = b * CP + c
                    acc = w[0] * crows_v[buf, r, pl.ds(0, LANES)]
                    for j in range(1, 8):
                        acc = acc + w[j] * crows_v[buf, r,
                                                   pl.ds(LANES * j, LANES)]
                    total = jnp.cumsum(acc)
                    plsc.store_compressed(
                        out_v.at[pl.ds(slot * CHC + r, LANES)], total,
                        mask=last_lane)

            @plsc.parallel_loop(0, n_red, unroll=1)
            def _(q):
                off = slot * CHC + q * LANES
                x = out_v[pl.ds(off, LANES)]
                out_v[pl.ds(off, LANES)] = 1.0 / (1.0 + jnp.exp(-x))

        fire(0, 0)
        fire(1, 1)

        @pl.loop(0, n_chunks - 2, step=2)
        def _(g):
            wait(0)
            compute(g, 0)
            fire(g + 2, 0)
            wait(1)
            compute(g + 1, 1)
            fire(g + 3, 1)

            @pl.when(lax.rem(g + 1, OB) == OB - 1)
            def _():
                pltpu.sync_copy(
                    out_v.at[pl.ds(0, OB * CHC)],
                    out_hbm.at[pl.ds(cbase + (g + 1 - (OB - 1)) * CHC,
                                     OB * CHC)])

        wait(0)
        compute(n_chunks - 2, 0)
        wait(1)
        compute(n_chunks - 1, 1)
        pltpu.sync_copy(
            out_v.at[pl.ds(0, OB * CHC)],
            out_hbm.at[pl.ds(cbase + (n_chunks - OB) * CHC, OB * CHC)])

    return w2v_kernel


def kernel(words, positive_contexts, negative_contexts, word_emb, context_emb):
    B = words.shape[0]
    P = positive_contexts.shape[1]
    N = negative_contexts.shape[1]
    CP = P + N
    cidx = jnp.concatenate([positive_contexts, negative_contexts],
                           axis=1).reshape(B * CP)
    out = _w2v_sc_call(B, CP)(word_emb, context_emb, words, cidx)
    out = out.reshape(B, CP)
    return out[:, :P], out[:, P:]


# speedup vs baseline: 1.0438x; 1.0131x over previous
import dataclasses
import functools

import jax
import jax.numpy as jnp
from jax import lax
from jax.experimental import pallas as pl
from jax.experimental.pallas import tpu as pltpu
from jax.experimental.pallas import tpu_sc as plsc

D = 128
LANES = 16
NC, NS = 2, 16
NW = NC * NS


@functools.lru_cache(maxsize=None)
def _w2v_sc_call(B, CP):
    rows_total = B * CP
    b_per_w = B // NW
    c_per_w = rows_total // NW
    CB = 8
    CHC = CB * CP
    n_chunks = b_per_w // CB
    n_red = CHC // LANES
    OB = 16
    mesh = plsc.VectorSubcoreMesh(core_axis_name="c", subcore_axis_name="s")
    cp = pltpu.CompilerParams()
    if "needs_layout_passes" in pltpu.CompilerParams.__dataclass_fields__:
        cp = dataclasses.replace(cp, needs_layout_passes=False)

    @functools.partial(
        pl.kernel,
        mesh=mesh,
        compiler_params=cp,
        out_type=jax.ShapeDtypeStruct((rows_total,), jnp.float32),
        scratch_types=[
            pltpu.VMEM((b_per_w,), jnp.int32),
            pltpu.VMEM((c_per_w,), jnp.int32),
            pltpu.VMEM((OB * CB, D), jnp.float32),
            pltpu.VMEM((2, CHC, D), jnp.float32),
            pltpu.VMEM((OB * CHC + LANES,), jnp.float32),
            pltpu.SemaphoreType.DMA,
            pltpu.SemaphoreType.DMA,
            pltpu.SemaphoreType.DMA,
            pltpu.SemaphoreType.DMA,
        ],
    )
    def w2v_kernel(wemb, cemb, widx_hbm, cidx_hbm, out_hbm,
                   widx_v, cidx_v, wrows_v, crows_v, out_v,
                   sem_w0, sem_w1, sem_c0, sem_c1):
        wid = lax.axis_index("s") * NC + lax.axis_index("c")
        wbase = wid * b_per_w
        cbase = wid * c_per_w
        pltpu.sync_copy(widx_hbm.at[pl.ds(wbase, b_per_w)], widx_v)
        pltpu.sync_copy(cidx_hbm.at[pl.ds(cbase, c_per_w)], cidx_v)
        last_lane = lax.iota(jnp.int32, LANES) == LANES - 1
        sems = ((sem_w0, sem_c0), (sem_w1, sem_c1))

        def fire(k, buf):
            sw, sc2 = sems[buf]
            pltpu.async_copy(
                cemb.at[cidx_v.at[pl.ds(k * CHC, CHC)]], crows_v.at[buf], sc2)

        def fire_words(blk):
            pltpu.async_copy(
                wemb.at[widx_v.at[pl.ds(blk * OB * CB, OB * CB)]], wrows_v,
                sem_w0)

        def wait_words():
            pltpu.make_async_copy(
                wemb.at[widx_v.at[pl.ds(0, OB * CB)]], wrows_v, sem_w0).wait()

        def wait(buf):
            sw, sc2 = sems[buf]
            pltpu.make_async_copy(
                cemb.at[cidx_v.at[pl.ds(0, CHC)]], crows_v.at[buf],
                sc2).wait()

        def compute(g, buf):
            slot = lax.rem(g, OB)
            for b in range(CB):
                w = [wrows_v[slot * CB + b, pl.ds(LANES * j, LANES)]
                     for j in range(8)]

                @plsc.parallel_loop(0, CP, unroll=1)
                def _(c):
                    r = b * CP + c
                    acc = w[0] * crows_v[buf, r, pl.ds(0, LANES)]
                    for j in range(1, 8):
                        acc = acc + w[j] * crows_v[buf, r,
                                                   pl.ds(LANES * j, LANES)]
                    total = jnp.cumsum(acc)
                    plsc.store_compressed(
                        out_v.at[pl.ds(slot * CHC + r, LANES)], total,
                        mask=last_lane)

            @plsc.parallel_loop(0, n_red, unroll=1)
            def _(q):
                off = slot * CHC + q * LANES
                x = out_v[pl.ds(off, LANES)]
                out_v[pl.ds(off, LANES)] = 1.0 / (1.0 + jnp.exp(-x))

        fire_words(0)
        fire(0, 0)
        fire(1, 1)
        wait_words()

        @pl.loop(0, n_chunks - 2, step=2)
        def _(g):
            wait(0)
            compute(g, 0)
            fire(g + 2, 0)
            wait(1)
            compute(g + 1, 1)
            fire(g + 3, 1)

            @pl.when(lax.rem(g + 1, OB) == OB - 1)
            def _():
                pltpu.sync_copy(
                    out_v.at[pl.ds(0, OB * CHC)],
                    out_hbm.at[pl.ds(cbase + (g + 1 - (OB - 1)) * CHC,
                                     OB * CHC)])
                blk = (g + 1) // OB

                @pl.when(blk < n_chunks // OB)
                def _():
                    pltpu.async_copy(
                        wemb.at[widx_v.at[pl.ds(blk * OB * CB, OB * CB)]],
                        wrows_v, sem_w0).wait()

        wait(0)
        compute(n_chunks - 2, 0)
        wait(1)
        compute(n_chunks - 1, 1)
        pltpu.sync_copy(
            out_v.at[pl.ds(0, OB * CHC)],
            out_hbm.at[pl.ds(cbase + (n_chunks - OB) * CHC, OB * CHC)])

    return w2v_kernel


def kernel(words, positive_contexts, negative_contexts, word_emb, context_emb):
    B = words.shape[0]
    P = positive_contexts.shape[1]
    N = negative_contexts.shape[1]
    CP = P + N
    cidx = jnp.concatenate([positive_contexts, negative_contexts],
                           axis=1).reshape(B * CP)
    out = _w2v_sc_call(B, CP)(word_emb, context_emb, words, cidx)
    out = out.reshape(B, CP)
    return out[:, :P], out[:, P:]


# speedup vs baseline: 1.0459x; 1.0020x over previous
import dataclasses
import functools

import jax
import jax.numpy as jnp
from jax import lax
from jax.experimental import pallas as pl
from jax.experimental.pallas import tpu as pltpu
from jax.experimental.pallas import tpu_sc as plsc

D = 128
LANES = 16
NC, NS = 2, 16
NW = NC * NS


@functools.lru_cache(maxsize=None)
def _w2v_sc_call(B, CP):
    rows_total = B * CP
    b_per_w = B // NW
    c_per_w = rows_total // NW
    CB = 8
    CHC = CB * CP
    n_chunks = b_per_w // CB
    n_red = CHC // LANES
    OB = 16
    mesh = plsc.VectorSubcoreMesh(core_axis_name="c", subcore_axis_name="s")
    cp = pltpu.CompilerParams()
    if "needs_layout_passes" in pltpu.CompilerParams.__dataclass_fields__:
        cp = dataclasses.replace(cp, needs_layout_passes=False)

    @functools.partial(
        pl.kernel,
        mesh=mesh,
        compiler_params=cp,
        out_type=jax.ShapeDtypeStruct((rows_total,), jnp.float32),
        scratch_types=[
            pltpu.VMEM((b_per_w,), jnp.int32),
            pltpu.VMEM((c_per_w,), jnp.int32),
            pltpu.VMEM((OB * CB, D), jnp.float32),
            pltpu.VMEM((2, CHC, D), jnp.float32),
            pltpu.VMEM((OB * CHC + LANES,), jnp.float32),
            pltpu.SemaphoreType.DMA,
            pltpu.SemaphoreType.DMA,
            pltpu.SemaphoreType.DMA,
            pltpu.SemaphoreType.DMA,
        ],
    )
    def w2v_kernel(wemb, cemb, widx_hbm, cidx_hbm, out_hbm,
                   widx_v, cidx_v, wrows_v, crows_v, out_v,
                   sem_w0, sem_w1, sem_c0, sem_c1):
        wid = lax.axis_index("s") * NC + lax.axis_index("c")
        wbase = wid * b_per_w
        cbase = wid * c_per_w
        pltpu.sync_copy(widx_hbm.at[pl.ds(wbase, b_per_w)], widx_v)
        pltpu.sync_copy(cidx_hbm.at[pl.ds(cbase, c_per_w)], cidx_v)
        last_lane = lax.iota(jnp.int32, LANES) == LANES - 1
        sems = ((sem_w0, sem_c0), (sem_w1, sem_c1))

        def fire(k, buf):
            sw, sc2 = sems[buf]
            pltpu.async_copy(
                cemb.at[cidx_v.at[pl.ds(k * CHC, CHC)]], crows_v.at[buf], sc2)

        def fire_words(blk):
            pltpu.async_copy(
                wemb.at[widx_v.at[pl.ds(blk * OB * CB, OB * CB)]], wrows_v,
                sem_w0)

        def wait_words():
            pltpu.make_async_copy(
                wemb.at[widx_v.at[pl.ds(0, OB * CB)]], wrows_v, sem_w0).wait()

        def wait(buf):
            sw, sc2 = sems[buf]
            pltpu.make_async_copy(
                cemb.at[cidx_v.at[pl.ds(0, CHC)]], crows_v.at[buf],
                sc2).wait()

        def compute(g, buf):
            slot = lax.rem(g, OB)
            for b in range(CB):
                w = [wrows_v[slot * CB + b, pl.ds(LANES * j, LANES)]
                     for j in range(8)]

                @plsc.parallel_loop(0, CP, unroll=1)
                def _(c):
                    r = b * CP + c
                    acc = w[0] * crows_v[buf, r, pl.ds(0, LANES)]
                    for j in range(1, 8):
                        acc = acc + w[j] * crows_v[buf, r,
                                                   pl.ds(LANES * j, LANES)]
                    total = jnp.cumsum(acc)
                    plsc.store_compressed(
                        out_v.at[pl.ds(slot * CHC + r, LANES)], total,
                        mask=last_lane)

            @plsc.parallel_loop(0, n_red, unroll=1)
            def _(q):
                off = slot * CHC + q * LANES
                x = out_v[pl.ds(off, LANES)]
                out_v[pl.ds(off, LANES)] = 1.0 / (1.0 + jnp.exp(-x))

        fire_words(0)
        fire(0, 0)
        fire(1, 1)
        wait_words()

        @pl.loop(0, n_chunks - 2, step=2)
        def _(g):
            wait(0)
            compute(g, 0)
            fire(g + 2, 0)
            wait(1)
            compute(g + 1, 1)
            fire(g + 3, 1)

            @pl.when(lax.rem(g + 1, OB) == OB - 1)
            def _():
                pltpu.sync_copy(
                    out_v.at[pl.ds(0, OB * CHC)],
                    out_hbm.at[pl.ds(cbase + (g + 1 - (OB - 1)) * CHC,
                                     OB * CHC)])
                blk = (g + 1) // OB + 1

                @pl.when(blk < n_chunks // OB)
                def _():
                    pltpu.async_copy(
                        wemb.at[widx_v.at[pl.ds(blk * OB * CB, OB * CB)]],
                        wrows_v, sem_w0).wait()

        wait(0)
        compute(n_chunks - 2, 0)
        wait(1)
        compute(n_chunks - 1, 1)
        pltpu.sync_copy(
            out_v.at[pl.ds(0, OB * CHC)],
            out_hbm.at[pl.ds(cbase + (n_chunks - OB) * CHC, OB * CHC)])

    return w2v_kernel


def kernel(words, positive_contexts, negative_contexts, word_emb, context_emb):
    B = words.shape[0]
    P = positive_contexts.shape[1]
    N = negative_contexts.shape[1]
    CP = P + N
    cidx = jnp.concatenate([positive_contexts, negative_contexts],
                           axis=1).reshape(B * CP)
    out = _w2v_sc_call(B, CP)(word_emb, context_emb, words, cidx)
    out = out.reshape(B, CP)
    return out[:, :P], out[:, P:]


# speedup vs baseline: 1.0795x; 1.0321x over previous
import dataclasses
import functools

import jax
import jax.numpy as jnp
from jax import lax
from jax.experimental import pallas as pl
from jax.experimental.pallas import tpu as pltpu
from jax.experimental.pallas import tpu_sc as plsc

D = 128
LANES = 16
NC, NS = 2, 16
NW = NC * NS


@functools.lru_cache(maxsize=None)
def _w2v_sc_call(B, CP):
    rows_total = B * CP
    b_per_w = B // NW
    c_per_w = rows_total // NW
    CB = 8
    CHC = CB * CP
    n_chunks = b_per_w // CB
    n_red = CHC // LANES
    OB = 16
    mesh = plsc.VectorSubcoreMesh(core_axis_name="c", subcore_axis_name="s")
    cp = pltpu.CompilerParams()
    if "needs_layout_passes" in pltpu.CompilerParams.__dataclass_fields__:
        cp = dataclasses.replace(cp, needs_layout_passes=False)

    @functools.partial(
        pl.kernel,
        mesh=mesh,
        compiler_params=cp,
        out_type=jax.ShapeDtypeStruct((rows_total,), jnp.float32),
        scratch_types=[
            pltpu.VMEM((b_per_w,), jnp.int32),
            pltpu.VMEM((c_per_w,), jnp.int32),
            pltpu.VMEM((2, CB, D), jnp.float32),
            pltpu.VMEM((2, CHC, D), jnp.float32),
            pltpu.VMEM((OB * CHC + LANES,), jnp.float32),
            pltpu.SemaphoreType.DMA,
            pltpu.SemaphoreType.DMA,
            pltpu.SemaphoreType.DMA,
            pltpu.SemaphoreType.DMA,
        ],
    )
    def w2v_kernel(wemb, cemb, widx_hbm, cidx_hbm, out_hbm,
                   widx_v, cidx_v, wrows_v, crows_v, out_v,
                   sem_w0, sem_w1, sem_c0, sem_c1):
        wid = lax.axis_index("s") * NC + lax.axis_index("c")
        wbase = wid * b_per_w
        cbase = wid * c_per_w
        pltpu.sync_copy(widx_hbm.at[pl.ds(wbase, b_per_w)], widx_v)
        pltpu.sync_copy(cidx_hbm.at[pl.ds(cbase, c_per_w)], cidx_v)
        last_lane = lax.iota(jnp.int32, LANES) == LANES - 1
        sems = ((sem_w0, sem_c0), (sem_w1, sem_c1))

        def fire(k, buf):
            sw, sc2 = sems[buf]
            pltpu.async_copy(
                wemb.at[widx_v.at[pl.ds(k * CB, CB)]], wrows_v.at[buf], sw)
            pltpu.async_copy(
                cemb.at[cidx_v.at[pl.ds(k * CHC, CHC)]], crows_v.at[buf], sc2)

        def wait(buf):
            sw, sc2 = sems[buf]
            pltpu.make_async_copy(
                wemb.at[widx_v.at[pl.ds(0, CB)]], wrows_v.at[buf], sw).wait()
            pltpu.make_async_copy(
                cemb.at[cidx_v.at[pl.ds(0, CHC)]], crows_v.at[buf],
                sc2).wait()

        def compute(g, buf):
            slot = lax.rem(g, OB)
            for b in range(CB):
                w = [wrows_v[buf, b, pl.ds(LANES * j, LANES)] for j in range(8)]

                @plsc.parallel_loop(0, CP, unroll=1)
                def _(c):
                    r = b * CP + c
                    acc = w[0] * crows_v[buf, r, pl.ds(0, LANES)]
                    for j in range(1, 8):
                        acc = acc + w[j] * crows_v[buf, r,
                                                   pl.ds(LANES * j, LANES)]
                    total = jnp.cumsum(acc)
                    plsc.store_compressed(
                        out_v.at[pl.ds(slot * CHC + r, LANES)], total,
                        mask=last_lane)

            @plsc.parallel_loop(0, n_red, unroll=1)
            def _(q):
                off = slot * CHC + q * LANES
                x = out_v[pl.ds(off, LANES)]
                out_v[pl.ds(off, LANES)] = 1.0 / (1.0 + jnp.exp(-x))

        fire(0, 0)
        fire(1, 1)

        @pl.loop(0, n_chunks - 2, step=2)
        def _(g):
            wait(0)
            compute(g, 0)
            fire(g + 2, 0)
            wait(1)
            compute(g + 1, 1)
            fire(g + 3, 1)

            @pl.when(lax.rem(g + 1, OB) == OB - 1)
            def _():
                pltpu.sync_copy(
                    out_v.at[pl.ds(0, OB * CHC)],
                    out_hbm.at[pl.ds(cbase + (g + 1 - (OB - 1)) * CHC,
                                     OB * CHC)])

        wait(0)
        compute(n_chunks - 2, 0)
        wait(1)
        compute(n_chunks - 1, 1)
        pltpu.sync_copy(
            out_v.at[pl.ds(0, OB * CHC)],
            out_hbm.at[pl.ds(cbase + (n_chunks - OB) * CHC, OB * CHC)])

    return w2v_kernel


def kernel(words, positive_contexts, negative_contexts, word_emb, context_emb):
    B = words.shape[0]
    P = positive_contexts.shape[1]
    N = negative_contexts.shape[1]
    CP = P + N
    cidx = jnp.concatenate([positive_contexts, negative_contexts],
                           axis=1).reshape(B * CP)
    out = _w2v_sc_call(B, CP)(word_emb, context_emb, words, cidx)
    out = out.reshape(B, CP)
    return out[:, :P], out[:, P:]
